# baseline (speedup 1.0000x reference)
import functools
import math

import jax
import jax.numpy as jnp
from jax import lax
from jax.experimental import pallas as pl
from jax.experimental.pallas import tpu as pltpu
from jax.experimental.pallas import tpu_sc as plsc

N_ATOMS = 10000
HIDDEN = 128
NFILT = 128
NINT = 6
NGAUSS = 50
CUTOFF = 4.7
MAXNB = 32
LOG2 = math.log(2.0)

BR = 512
NBLK = (N_ATOMS + BR - 1) // BR
NPAD = NBLK * BR
EPAD = NPAD * MAXNB
CUT2 = CUTOFF * CUTOFF
BIGV = 1e30
FARPOS = 1.0e6

NC = 2
NS = 16
NW = NC * NS
RPW = NPAD // NW
NCHUNK = NPAD // 16
CAND = 96


def _ssp(x):
    return jnp.maximum(x, 0.0) + jnp.log1p(jnp.exp(-jnp.abs(x))) - LOG2


def _d2_body(pr_ref, prs_ref, pct_ref, csq_ref, out_ref):
    i = pl.program_id(0)
    g = jnp.dot(pr_ref[...], pct_ref[...], preferred_element_type=jnp.float32)
    d2 = prs_ref[...] + csq_ref[...] - 2.0 * g
    rowg = jax.lax.broadcasted_iota(jnp.int32, (BR, NPAD), 0) + i * BR
    colg = jax.lax.broadcasted_iota(jnp.int32, (BR, NPAD), 1)
    bad = (rowg == colg) | (d2 > CUT2)
    out_ref[...] = jnp.where(bad, BIGV, d2)


def _d2mask(posp, prs, pct, csq):
    return pl.pallas_call(
        _d2_body,
        grid=(NBLK,),
        in_specs=[
            pl.BlockSpec((BR, 8), lambda i: (i, 0)),
            pl.BlockSpec((BR, 1), lambda i: (i, 0)),
            pl.BlockSpec((8, NPAD), lambda i: (0, 0)),
            pl.BlockSpec((1, NPAD), lambda i: (0, 0)),
        ],
        out_specs=pl.BlockSpec((BR, NPAD), lambda i: (i, 0)),
        out_shape=jax.ShapeDtypeStruct((NPAD, NPAD), jnp.float32),
    )(posp, prs, pct, csq)


_sc_mesh = plsc.VectorSubcoreMesh(core_axis_name="c", subcore_axis_name="s")


@functools.partial(
    pl.kernel,
    mesh=_sc_mesh,
    compiler_params=pltpu.CompilerParams(needs_layout_passes=False),
    out_type=[
        jax.ShapeDtypeStruct((NPAD, MAXNB), jnp.float32),
        jax.ShapeDtypeStruct((NPAD, MAXNB), jnp.int32),
    ],
    scratch_types=[
        pltpu.VMEM((NPAD,), jnp.float32),
        pltpu.VMEM((CAND,), jnp.float32),
        pltpu.VMEM((CAND,), jnp.int32),
        pltpu.VMEM((MAXNB,), jnp.float32),
        pltpu.VMEM((MAXNB,), jnp.int32),
    ],
)
def _select32(d_hbm, outd_hbm, outi_hbm, row_v, cd_v, ci_v, od_v, oi_v):
    wid = lax.axis_index("s") * NC + lax.axis_index("c")
    base = wid * RPW
    nv = CAND // 16

    for k0 in range(CAND // 16):
        ci_v[pl.ds(k0 * 16, 16)] = jnp.zeros((16,), jnp.int32)

    def row_body(r, carry):
        pltpu.sync_copy(d_hbm.at[base + r], row_v)
        for k in range(nv):
            cd_v[pl.ds(k * 16, 16)] = jnp.full((16,), BIGV, jnp.float32)

        def chunk_body(c, cnt):
            v = row_v[pl.ds(c * 16, 16)]
            m = v < BIGV
            idx = lax.iota(jnp.int32, 16) + c * 16
            pref = jnp.cumsum(m.astype(jnp.int32))
            ppos = cnt + pref - 1
            plsc.store_scatter(cd_v, [ppos], v, mask=m)
            plsc.store_scatter(ci_v, [ppos], idx, mask=m)
            return cnt + pref[15]

        lax.fori_loop(0, NCHUNK, chunk_body, 0)

        def ext16(acc, _k0):
            def b(k, carry):
                acc_d, acc_i = carry
                vals = [cd_v[pl.ds(j * 16, 16)] for j in range(nv)]
                vm = vals[0]
                for j in range(1, nv):
                    vm = jnp.minimum(vm, vals[j])
                mval = jnp.min(vm)
                imin = jnp.int32(2 ** 30)
                for j in range(nv):
                    cand = jnp.where(vals[j] == mval, ci_v[pl.ds(j * 16, 16)],
                                     jnp.int32(2 ** 30))
                    imin = jnp.minimum(imin, jnp.min(cand))
                for j in range(nv):
                    zap = (vals[j] == mval) & (
                        ci_v[pl.ds(j * 16, 16)] == imin)
                    cd_v[pl.ds(j * 16, 16)] = jnp.where(zap, BIGV, vals[j])
                sel = lax.iota(jnp.int32, 16) == k
                acc_d = jnp.where(sel, mval, acc_d)
                acc_i = jnp.where(sel, imin, acc_i)
                return acc_d, acc_i
            return lax.fori_loop(0, 16, b, acc)

        zf = jnp.zeros((16,), jnp.float32)
        zi = jnp.zeros((16,), jnp.int32)
        d0, i0 = ext16((zf, zi), 0)
        d1, i1 = ext16((zf, zi), 1)
        od_v[pl.ds(0, 16)] = d0
        od_v[pl.ds(16, 16)] = d1
        oi_v[pl.ds(0, 16)] = i0
        oi_v[pl.ds(16, 16)] = i1
        pltpu.sync_copy(od_v, outd_hbm.at[base + r])
        pltpu.sync_copy(oi_v, outi_hbm.at[base + r])
        return carry

    lax.fori_loop(0, RPW, row_body, 0)


def _block_body(d2_ref, xlg_ref, h_ref,
                bw1_ref, bb1_ref, bw2_ref, bb2_ref,
                cw2_ref, cb2_ref, lw_ref, lb_ref, out_ref):
    x = d2_ref[...]
    nr = BR * MAXNB // 128
    xb = jnp.broadcast_to(x.reshape(nr, 1, 128),
                          (nr, 128, 128)).reshape(BR * MAXNB, 128)
    lane = jax.lax.broadcasted_iota(jnp.int32, (BR * MAXNB, 128), 1)
    sub = jax.lax.broadcasted_iota(jnp.int32, (BR * MAXNB, 128), 0) % 128
    d2col = jnp.sum(jnp.where(lane == sub, xb, 0.0), axis=1, keepdims=True)
    d = jnp.sqrt(d2col)
    offset = jax.lax.broadcasted_iota(
        jnp.int32, (1, NGAUSS), 1).astype(jnp.float32) * (CUTOFF / (NGAUSS - 1))
    coeff = -0.5 / (CUTOFF / (NGAUSS - 1)) ** 2
    ea = jnp.exp(coeff * (d - offset) ** 2)
    a = _ssp(jnp.dot(ea, bw1_ref[...], preferred_element_type=jnp.float32)
             + bb1_ref[...])
    w = jnp.dot(a, bw2_ref[...], preferred_element_type=jnp.float32) \
        + bb2_ref[...]
    cutmask = (d <= CUTOFF).astype(jnp.float32)
    c = 0.5 * (jnp.cos(d * (math.pi / CUTOFF)) + 1.0) * cutmask
    msg = xlg_ref[...].astype(jnp.float32) * (w * c)
    agg = jnp.sum(msg.reshape(BR, MAXNB, NFILT), axis=1)
    v = _ssp(jnp.dot(agg, cw2_ref[...], preferred_element_type=jnp.float32)
             + cb2_ref[...])
    out_ref[...] = h_ref[...] + jnp.dot(
        v, lw_ref[...], preferred_element_type=jnp.float32) + lb_ref[...]


def _full2d(shape):
    return pl.BlockSpec(shape, lambda i: (0, 0))


def _interaction_block(h, d2e, xlg, bw1, bb1, bw2, bb2, cw2, cb2, lw, lb):
    return pl.pallas_call(
        _block_body,
        grid=(NBLK,),
        in_specs=[
            pl.BlockSpec((BR * MAXNB // 128, 128), lambda i: (i, 0)),
            pl.BlockSpec((BR * MAXNB, NFILT), lambda i: (i, 0)),
            pl.BlockSpec((BR, HIDDEN), lambda i: (i, 0)),
            _full2d((NGAUSS, NFILT)),
            pl.BlockSpec((NFILT,), lambda i: (0,)),
            _full2d((NFILT, NFILT)),
            pl.BlockSpec((NFILT,), lambda i: (0,)),
            _full2d((NFILT, HIDDEN)),
            pl.BlockSpec((HIDDEN,), lambda i: (0,)),
            _full2d((HIDDEN, HIDDEN)),
            pl.BlockSpec((HIDDEN,), lambda i: (0,)),
        ],
        out_specs=pl.BlockSpec((BR, HIDDEN), lambda i: (i, 0)),
        out_shape=jax.ShapeDtypeStruct((NPAD, HIDDEN), jnp.float32),
    )(d2e, xlg, h, bw1, bb1, bw2, bb2, cw2, cb2, lw, lb)


def kernel(z, pos, emb, mlp_w1, mlp_b1, mlp_w2, mlp_b2, cfc_w1, cfc_w2,
           cfc_b2, lin_w, lin_b, out_w1, out_b1, out_w2, out_b2):
    pad = NPAD - N_ATOMS
    posp = jnp.concatenate(
        [pos, jnp.full((pad, 3), FARPOS, jnp.float32)], axis=0)
    posp = jnp.pad(posp, ((0, 0), (0, 5)))
    sq = jnp.sum(posp * posp, axis=1)
    d_mat = _d2mask(posp, sq[:, None], posp.T, sq[None, :])
    nd2, nidx = _select32(d_mat)
    d2e = nd2.reshape(EPAD // 128, 128)
    src_flat = nidx.reshape(-1)
    h_p = jnp.pad(emb[z], ((0, pad), (0, 0)))
    for i in range(NINT):
        xl = h_p @ cfc_w1[i]
        xlg = xl.astype(jnp.bfloat16)[src_flat]
        h_p = _interaction_block(h_p, d2e, xlg, mlp_w1[i], mlp_b1[i],
                                 mlp_w2[i], mlp_b2[i], cfc_w2[i], cfc_b2[i],
                                 lin_w[i], lin_b[i])
    h = h_p[:N_ATOMS]
    h = _ssp(h @ out_w1 + out_b1)
    h = h @ out_w2 + out_b2
    return jnp.sum(h, axis=0)


# speedup vs baseline: 1.0845x; 1.0845x over previous
import functools
import math

import jax
import jax.numpy as jnp
from jax import lax
from jax.experimental import pallas as pl
from jax.experimental.pallas import tpu as pltpu
from jax.experimental.pallas import tpu_sc as plsc

N_ATOMS = 10000
HIDDEN = 128
NFILT = 128
NINT = 6
NGAUSS = 50
CUTOFF = 4.7
MAXNB = 32
LOG2 = math.log(2.0)

BR = 256
NBLK = (N_ATOMS + BR - 1) // BR
NPAD = NBLK * BR
EPAD = NPAD * MAXNB
CUT2 = CUTOFF * CUTOFF
BIGV = 1e30
FARPOS = 1.0e6

NC = 2
NS = 16
NW = NC * NS
RPW = NPAD // NW
NCHUNK = NPAD // 16
CAND = 96


def _ssp(x):
    return jnp.maximum(x, 0.0) + jnp.log1p(jnp.exp(-jnp.abs(x))) - LOG2


def _d2_body(pr_ref, prs_ref, pct_ref, csq_ref, out_ref):
    i = pl.program_id(0)
    g = jnp.dot(pr_ref[...], pct_ref[...], preferred_element_type=jnp.float32)
    d2 = prs_ref[...] + csq_ref[...] - 2.0 * g
    rowg = jax.lax.broadcasted_iota(jnp.int32, (BR, NPAD), 0) + i * BR
    colg = jax.lax.broadcasted_iota(jnp.int32, (BR, NPAD), 1)
    bad = (rowg == colg) | (d2 > CUT2)
    out_ref[...] = jnp.where(bad, BIGV, d2)


def _d2mask(posp, prs, pct, csq):
    return pl.pallas_call(
        _d2_body,
        grid=(NBLK,),
        in_specs=[
            pl.BlockSpec((BR, 8), lambda i: (i, 0)),
            pl.BlockSpec((BR, 1), lambda i: (i, 0)),
            pl.BlockSpec((8, NPAD), lambda i: (0, 0)),
            pl.BlockSpec((1, NPAD), lambda i: (0, 0)),
        ],
        out_specs=pl.BlockSpec((BR, NPAD), lambda i: (i, 0)),
        out_shape=jax.ShapeDtypeStruct((NPAD, NPAD), jnp.float32),
    )(posp, prs, pct, csq)


_sc_mesh = plsc.VectorSubcoreMesh(core_axis_name="c", subcore_axis_name="s")


@functools.partial(
    pl.kernel,
    mesh=_sc_mesh,
    compiler_params=pltpu.CompilerParams(needs_layout_passes=False),
    out_type=[
        jax.ShapeDtypeStruct((NPAD, MAXNB), jnp.float32),
        jax.ShapeDtypeStruct((NPAD, MAXNB), jnp.int32),
    ],
    scratch_types=[
        pltpu.VMEM((NPAD,), jnp.float32),
        pltpu.VMEM((CAND,), jnp.float32),
        pltpu.VMEM((CAND,), jnp.int32),
        pltpu.VMEM((MAXNB,), jnp.float32),
        pltpu.VMEM((MAXNB,), jnp.int32),
    ],
)
def _select32(d_hbm, outd_hbm, outi_hbm, row_v, cd_v, ci_v, od_v, oi_v):
    wid = lax.axis_index("s") * NC + lax.axis_index("c")
    base = wid * RPW
    nv = CAND // 16

    for k0 in range(CAND // 16):
        ci_v[pl.ds(k0 * 16, 16)] = jnp.zeros((16,), jnp.int32)

    def row_body(r, carry):
        pltpu.sync_copy(d_hbm.at[base + r], row_v)
        for k in range(nv):
            cd_v[pl.ds(k * 16, 16)] = jnp.full((16,), BIGV, jnp.float32)

        def chunk_body(c, cnt):
            v = row_v[pl.ds(c * 16, 16)]
            m = v < BIGV
            idx = lax.iota(jnp.int32, 16) + c * 16
            pref = jnp.cumsum(m.astype(jnp.int32))
            ppos = cnt + pref - 1
            plsc.store_scatter(cd_v, [ppos], v, mask=m)
            plsc.store_scatter(ci_v, [ppos], idx, mask=m)
            return cnt + pref[15]

        lax.fori_loop(0, NCHUNK, chunk_body, 0)

        def ext16(acc, _k0):
            def b(k, carry):
                acc_d, acc_i = carry
                vals = [cd_v[pl.ds(j * 16, 16)] for j in range(nv)]
                vm = vals[0]
                for j in range(1, nv):
                    vm = jnp.minimum(vm, vals[j])
                mval = jnp.min(vm)
                imin = jnp.int32(2 ** 30)
                for j in range(nv):
                    cand = jnp.where(vals[j] == mval, ci_v[pl.ds(j * 16, 16)],
                                     jnp.int32(2 ** 30))
                    imin = jnp.minimum(imin, jnp.min(cand))
                for j in range(nv):
                    zap = (vals[j] == mval) & (
                        ci_v[pl.ds(j * 16, 16)] == imin)
                    cd_v[pl.ds(j * 16, 16)] = jnp.where(zap, BIGV, vals[j])
                sel = lax.iota(jnp.int32, 16) == k
                acc_d = jnp.where(sel, mval, acc_d)
                acc_i = jnp.where(sel, imin, acc_i)
                return acc_d, acc_i
            return lax.fori_loop(0, 16, b, acc)

        zf = jnp.zeros((16,), jnp.float32)
        zi = jnp.zeros((16,), jnp.int32)
        d0, i0 = ext16((zf, zi), 0)
        d1, i1 = ext16((zf, zi), 1)
        od_v[pl.ds(0, 16)] = d0
        od_v[pl.ds(16, 16)] = d1
        oi_v[pl.ds(0, 16)] = i0
        oi_v[pl.ds(16, 16)] = i1
        pltpu.sync_copy(od_v, outd_hbm.at[base + r])
        pltpu.sync_copy(oi_v, outi_hbm.at[base + r])
        return carry

    lax.fori_loop(0, RPW, row_body, 0)


def _block_body(d2_ref, xlg_ref, h_ref,
                bw1_ref, bb1_ref, bw2_ref, bb2_ref,
                cw2_ref, cb2_ref, lw_ref, lb_ref, out_ref):
    x = d2_ref[...]
    nr = BR * MAXNB // 128
    xb = jnp.broadcast_to(x.reshape(nr, 1, 128),
                          (nr, 128, 128)).reshape(BR * MAXNB, 128)
    lane = jax.lax.broadcasted_iota(jnp.int32, (BR * MAXNB, 128), 1)
    sub = jax.lax.broadcasted_iota(jnp.int32, (BR * MAXNB, 128), 0) % 128
    d2col = jnp.sum(jnp.where(lane == sub, xb, 0.0), axis=1, keepdims=True)
    d = jnp.sqrt(d2col)
    offset = jax.lax.broadcasted_iota(
        jnp.int32, (1, NGAUSS), 1).astype(jnp.float32) * (CUTOFF / (NGAUSS - 1))
    coeff = -0.5 / (CUTOFF / (NGAUSS - 1)) ** 2
    ea = jnp.exp(coeff * (d - offset) ** 2)
    a = _ssp(jnp.dot(ea, bw1_ref[...], preferred_element_type=jnp.float32)
             + bb1_ref[...])
    w = jnp.dot(a, bw2_ref[...], preferred_element_type=jnp.float32) \
        + bb2_ref[...]
    cutmask = (d <= CUTOFF).astype(jnp.float32)
    c = 0.5 * (jnp.cos(d * (math.pi / CUTOFF)) + 1.0) * cutmask
    msg = xlg_ref[...].astype(jnp.float32) * (w * c)
    agg = jnp.sum(msg.reshape(BR, MAXNB, NFILT), axis=1)
    v = _ssp(jnp.dot(agg, cw2_ref[...], preferred_element_type=jnp.float32)
             + cb2_ref[...])
    out_ref[...] = h_ref[...] + jnp.dot(
        v, lw_ref[...], preferred_element_type=jnp.float32) + lb_ref[...]


def _full2d(shape):
    return pl.BlockSpec(shape, lambda i: (0, 0))


def _interaction_block(h, d2e, xlg, bw1, bb1, bw2, bb2, cw2, cb2, lw, lb):
    return pl.pallas_call(
        _block_body,
        grid=(NBLK,),
        in_specs=[
            pl.BlockSpec((BR * MAXNB // 128, 128), lambda i: (i, 0)),
            pl.BlockSpec((BR * MAXNB, NFILT), lambda i: (i, 0)),
            pl.BlockSpec((BR, HIDDEN), lambda i: (i, 0)),
            _full2d((NGAUSS, NFILT)),
            pl.BlockSpec((NFILT,), lambda i: (0,)),
            _full2d((NFILT, NFILT)),
            pl.BlockSpec((NFILT,), lambda i: (0,)),
            _full2d((NFILT, HIDDEN)),
            pl.BlockSpec((HIDDEN,), lambda i: (0,)),
            _full2d((HIDDEN, HIDDEN)),
            pl.BlockSpec((HIDDEN,), lambda i: (0,)),
        ],
        out_specs=pl.BlockSpec((BR, HIDDEN), lambda i: (i, 0)),
        out_shape=jax.ShapeDtypeStruct((NPAD, HIDDEN), jnp.float32),
    )(d2e, xlg, h, bw1, bb1, bw2, bb2, cw2, cb2, lw, lb)


def kernel(z, pos, emb, mlp_w1, mlp_b1, mlp_w2, mlp_b2, cfc_w1, cfc_w2,
           cfc_b2, lin_w, lin_b, out_w1, out_b1, out_w2, out_b2):
    pad = NPAD - N_ATOMS
    posp = jnp.concatenate(
        [pos, jnp.full((pad, 3), FARPOS, jnp.float32)], axis=0)
    posp = jnp.pad(posp, ((0, 0), (0, 5)))
    sq = jnp.sum(posp * posp, axis=1)
    d_mat = _d2mask(posp, sq[:, None], posp.T, sq[None, :])
    nd2, nidx = _select32(d_mat)
    d2e = nd2.reshape(EPAD // 128, 128)
    src_flat = nidx.reshape(-1)
    h_p = jnp.pad(emb[z], ((0, pad), (0, 0)))
    for i in range(NINT):
        xl = h_p @ cfc_w1[i]
        xlg = xl.astype(jnp.bfloat16)[src_flat]
        h_p = _interaction_block(h_p, d2e, xlg, mlp_w1[i], mlp_b1[i],
                                 mlp_w2[i], mlp_b2[i], cfc_w2[i], cfc_b2[i],
                                 lin_w[i], lin_b[i])
    h = h_p[:N_ATOMS]
    h = _ssp(h @ out_w1 + out_b1)
    h = h @ out_w2 + out_b2
    return jnp.sum(h, axis=0)
